# baseline (speedup 1.0000x reference)
import jax
import jax.numpy as jnp
from jax import lax
from jax.experimental import pallas as pl
from jax.experimental.pallas import tpu as pltpu
from jax.experimental.pallas import tpu_sc as plsc

N_NODES = 10000
N_EDGES = 320000
D_FEAT = 128
CONV_H = 128
LIN_H = 64
NUM_CLASSES = 10
NUM_GRAPHS = 64

_NC = 2
_NS = 16
_NW = _NC * _NS
_EPT = N_EDGES // _NW
_CH = 80
_NCHUNK = N_EDGES // _CH
_CPT = _NCHUNK // _NS
_NBUF = 5
_HF = CONV_H // 2
_GRP = _CH // 16
_PR = 624
_PCH = 48


def _deg_body(ei_hbm, ew_hbm, out_hbm, col_v, w_v, deg_v):
    cid = lax.axis_index("c")
    sid = lax.axis_index("s")
    wid = sid * _NC + cid
    base = wid * _EPT
    pltpu.sync_copy(ei_hbm.at[1].at[pl.ds(base, _EPT)], col_v)
    pltpu.sync_copy(ew_hbm.at[pl.ds(base, _EPT)], w_v)

    def zero(i, _):
        deg_v[pl.ds(i * 16, 16)] = jnp.zeros((16,), jnp.float32)
        return 0

    lax.fori_loop(0, N_NODES // 16, zero, 0)

    def acc(i, _):
        sl = pl.ds(i * 16, 16)
        plsc.addupdate_scatter(deg_v, [col_v[sl]], w_v[sl])
        return 0

    lax.fori_loop(0, _EPT // 16, acc, 0)
    pltpu.sync_copy(deg_v, out_hbm.at[wid])


def _deg_partials(ei, ew):
    mesh = plsc.VectorSubcoreMesh(core_axis_name="c", subcore_axis_name="s")
    return pl.kernel(
        _deg_body,
        out_type=jax.ShapeDtypeStruct((_NW, N_NODES), jnp.float32),
        mesh=mesh,
        compiler_params=pltpu.CompilerParams(needs_layout_passes=False,
                                             use_tc_tiling_on_sc=False),
        scratch_types=[
            pltpu.VMEM((_EPT,), jnp.int32),
            pltpu.VMEM((_EPT,), jnp.float32),
            pltpu.VMEM((N_NODES,), jnp.float32),
        ],
    )(ei, ew)


def _mm_body(x_ref, w_ref, h_ref):
    h = jnp.dot(x_ref[...], w_ref[...], preferred_element_type=jnp.float32)
    h_ref[0] = h[:, :_HF]
    h_ref[1] = h[:, _HF:]


def _tc_mm(x, w1t):
    nb = 10
    blk = N_NODES // nb
    return pl.pallas_call(
        _mm_body,
        grid=(nb,),
        in_specs=[
            pl.BlockSpec((blk, D_FEAT), lambda i: (i, 0)),
            pl.BlockSpec((D_FEAT, CONV_H), lambda i: (0, 0)),
        ],
        out_specs=pl.BlockSpec((_NC, blk, _HF), lambda i: (0, i, 0)),
        out_shape=jax.ShapeDtypeStruct((_NC, N_NODES, _HF), jnp.float32),
    )(x, w1t)


def _dinv_body(degp_ref, dinv_ref):
    deg = jnp.sum(degp_ref[...], axis=0, keepdims=True) + 1.0
    dinv_ref[...] = lax.rsqrt(deg)


def _tc_dinv(degp):
    return pl.pallas_call(
        _dinv_body,
        out_shape=jax.ShapeDtypeStruct((1, N_NODES), jnp.float32),
    )(degp)


def _msg_body(h_hbm, row_hbm, col_hbm, ew_hbm, dinv_hbm, zero_hbm, batch_hbm,
              b1_hbm, pool_hbm, dinv_v, nrm_v, bv, accg, b1v, *rest):
    n = _NBUF
    ridx = rest[0:n]
    cidx = rest[n:2 * n]
    wch = rest[2 * n:3 * n]
    rows = rest[3 * n:4 * n]
    esem = rest[4 * n:5 * n]
    gsem = rest[5 * n:6 * n]
    ssem = rest[6 * n:7 * n]
    accum = rest[7 * n]

    cid = lax.axis_index("c")
    sid = lax.axis_index("s")
    cbase = sid * _CPT
    hsrc = h_hbm.at[cid]

    pltpu.sync_copy(dinv_hbm.at[0], dinv_v)
    pltpu.sync_copy(b1_hbm.at[pl.ds(cid * _HF, _HF)], b1v)

    @pl.when(sid == 0)
    def _():
        pltpu.sync_copy(zero_hbm, accum)

    plsc.subcore_barrier()

    def stage(c, s):
        pltpu.async_copy(row_hbm.at[c], ridx[s], esem[s])
        pltpu.async_copy(col_hbm.at[c], cidx[s], esem[s])
        pltpu.async_copy(ew_hbm.at[c], wch[s], esem[s])

    def ewait(s):
        pltpu.make_async_copy(row_hbm.at[0], ridx[s], esem[s]).wait()
        pltpu.make_async_copy(col_hbm.at[0], cidx[s], esem[s]).wait()
        pltpu.make_async_copy(ew_hbm.at[0], wch[s], esem[s]).wait()

    def gstart(s):
        pltpu.async_copy(hsrc.at[ridx[s]], rows[s], gsem[s])

    def gwait(s):
        pltpu.make_async_copy(hsrc.at[pl.ds(0, _CH)], rows[s],
                              gsem[s]).wait()

    def swait(s):
        pltpu.make_async_copy(rows[s], accum.at[pl.ds(0, _CH)],
                              ssem[s]).wait()

    def compute_scatter(s):
        for k in range(_GRP):
            sl = pl.ds(k * 16, 16)
            dr = plsc.load_gather(dinv_v, [ridx[s][sl]])
            dc = plsc.load_gather(dinv_v, [cidx[s][sl]])
            nrm_v[sl] = dr * wch[s][sl] * dc

        def scale(g, _):
            nrm16 = nrm_v[pl.ds(g * 16, 16)]
            for i16 in range(16):
                ei = g * 16 + i16
                sc = nrm16[i16]
                for f in range(_HF // 16):
                    fs = pl.ds(f * 16, 16)
                    rows[s][ei, fs] = rows[s][ei, fs] * sc
            return 0

        lax.fori_loop(0, _GRP, scale, 0)
        pltpu.async_copy(rows[s], accum.at[cidx[s]], ssem[s], add=True)

    for s in range(_NBUF - 1):
        stage(cbase + s, s)
    for s in range(_NBUF - 2):
        ewait(s)
        gstart(s)

    def outer(t, _):
        for b in range(n):
            j = t * n + b
            gwait(b)
            compute_scatter(b)
            s4 = (b + 4) % n

            @pl.when(j + 4 < _CPT)
            def _():
                @pl.when(j >= 1)
                def _():
                    swait(s4)

                stage(cbase + j + 4, s4)

            s3 = (b + 3) % n

            @pl.when(j + 3 < _CPT)
            def _():
                ewait(s3)
                gstart(s3)

        return 0

    lax.fori_loop(0, _CPT // n, outer, 0)

    for b in range(n):
        swait(b)

    plsc.subcore_barrier()

    def zacc(g, _):
        for f in range(_HF // 16):
            accg[g, pl.ds(f * 16, 16)] = jnp.full((16,), -jnp.inf,
                                                  jnp.float32)
        return 0

    lax.fori_loop(0, NUM_GRAPHS, zacc, 0)

    rb = sid * _PR
    abuf = rows[0]
    hbuf = rows[1]

    def pchunk(c, _):
        base = rb + c * _PCH
        sl = pl.ds(base, _PCH)
        pltpu.sync_copy(accum.at[sl], abuf.at[pl.ds(0, _PCH)])
        pltpu.sync_copy(hsrc.at[sl], hbuf.at[pl.ds(0, _PCH)])
        pltpu.sync_copy(batch_hbm.at[sl], bv)

        def pgrp(t, _):
            o16 = t * 16
            d16 = dinv_v[pl.ds(base + o16, 16)]
            b16 = bv[pl.ds(o16, 16)]
            for i in range(16):
                r = o16 + i
                dsc = d16[i]
                gid = b16[i]
                for f in range(_HF // 16):
                    fs = pl.ds(f * 16, 16)
                    v = abuf[r, fs] + (dsc * dsc) * hbuf[r, fs] + b1v[fs]
                    v = jnp.maximum(v, 0.0)
                    accg[gid, fs] = jnp.maximum(accg[gid, fs], v)
            return 0

        lax.fori_loop(0, _PCH // 16, pgrp, 0)
        return 0

    lax.fori_loop(0, _PR // _PCH, pchunk, 0)

    @pl.when(sid == _NS - 1)
    def _():
        tl = pl.ds(N_NODES - 16, 16)
        pltpu.sync_copy(accum.at[tl], abuf.at[pl.ds(0, 16)])
        pltpu.sync_copy(hsrc.at[tl], hbuf.at[pl.ds(0, 16)])
        pltpu.sync_copy(batch_hbm.at[tl], bv.at[pl.ds(0, 16)])
        d16 = dinv_v[tl]
        b16 = bv[pl.ds(0, 16)]
        for i in range(16):
            dsc = d16[i]
            gid = b16[i]
            for f in range(_HF // 16):
                fs = pl.ds(f * 16, 16)
                v = abuf[i, fs] + (dsc * dsc) * hbuf[i, fs] + b1v[fs]
                v = jnp.maximum(v, 0.0)
                accg[gid, fs] = jnp.maximum(accg[gid, fs], v)

    pltpu.sync_copy(accg, pool_hbm.at[cid].at[sid])


def _msg_pool(h2, row2d, col2d, ew2d, dinv, zeros, batch, b1):
    mesh = plsc.VectorSubcoreMesh(core_axis_name="c", subcore_axis_name="s")
    scratch = [
        pltpu.VMEM((N_NODES,), jnp.float32),
        pltpu.VMEM((_CH,), jnp.float32),
        pltpu.VMEM((_PCH,), jnp.int32),
        pltpu.VMEM((NUM_GRAPHS, _HF), jnp.float32),
        pltpu.VMEM((_HF,), jnp.float32),
    ]
    scratch += [pltpu.VMEM((_CH,), jnp.int32) for _ in range(_NBUF)]
    scratch += [pltpu.VMEM((_CH,), jnp.int32) for _ in range(_NBUF)]
    scratch += [pltpu.VMEM((_CH,), jnp.float32) for _ in range(_NBUF)]
    scratch += [pltpu.VMEM((_CH, _HF), jnp.float32) for _ in range(_NBUF)]
    scratch += [pltpu.SemaphoreType.DMA for _ in range(3 * _NBUF)]
    scratch += [pltpu.VMEM_SHARED((N_NODES, _HF), jnp.float32)]
    return pl.kernel(
        _msg_body,
        out_type=jax.ShapeDtypeStruct((_NC, _NS, NUM_GRAPHS, _HF),
                                      jnp.float32),
        mesh=mesh,
        compiler_params=pltpu.CompilerParams(needs_layout_passes=False,
                                             use_tc_tiling_on_sc=False),
        scratch_types=scratch,
    )(h2, row2d, col2d, ew2d, dinv, zeros, batch, b1)


def _head_body(pm_ref, l1wt_ref, l1b_ref, l2wt_ref, l2b_ref, out_ref):
    p0 = jnp.max(pm_ref[0], axis=0)
    p1 = jnp.max(pm_ref[1], axis=0)
    p = jnp.concatenate([p0, p1], axis=-1)
    z = jnp.dot(p, l1wt_ref[...], preferred_element_type=jnp.float32)
    z = jnp.maximum(z + l1b_ref[...], 0.0)
    out_ref[...] = (jnp.dot(z, l2wt_ref[...],
                            preferred_element_type=jnp.float32)
                    + l2b_ref[...])


def _tc_head(pm, l1wt, l1b, l2wt, l2b):
    return pl.pallas_call(
        _head_body,
        out_shape=jax.ShapeDtypeStruct((NUM_GRAPHS, NUM_CLASSES),
                                       jnp.float32),
    )(pm, l1wt, l1b, l2wt, l2b)


def kernel(x, edge_index, edge_weight, batch, W1, b1, lin1_w, lin1_b,
           lin2_w, lin2_b):
    ei = edge_index.astype(jnp.int32)
    ew = jnp.ravel(edge_weight).astype(jnp.float32)
    x = x.astype(jnp.float32)

    degp = _deg_partials(ei, ew)
    h2 = _tc_mm(x, W1.T)
    dinv = _tc_dinv(degp)
    zeros = jnp.zeros((N_NODES, _HF), jnp.float32)
    pm = _msg_pool(h2, ei[0].reshape(_NCHUNK, _CH),
                   ei[1].reshape(_NCHUNK, _CH),
                   ew.reshape(_NCHUNK, _CH), dinv, zeros,
                   batch.astype(jnp.int32), b1)
    return _tc_head(pm, lin1_w.T, lin1_b.reshape(1, LIN_H),
                    lin2_w.T, lin2_b.reshape(1, NUM_CLASSES))


# speedup vs baseline: 1.0004x; 1.0004x over previous
import jax
import jax.numpy as jnp
from jax import lax
from jax.experimental import pallas as pl
from jax.experimental.pallas import tpu as pltpu
from jax.experimental.pallas import tpu_sc as plsc

N_NODES = 10000
N_EDGES = 320000
D_FEAT = 128
CONV_H = 128
LIN_H = 64
NUM_CLASSES = 10
NUM_GRAPHS = 64

_NC = 2
_NS = 16
_NW = _NC * _NS
_EPT = N_EDGES // _NW
_CH = 80
_NCHUNK = N_EDGES // _CH
_CPT = _NCHUNK // _NS
_NBUF = 5
_HF = CONV_H // 2
_GRP = _CH // 16
_PR = 624
_PCH = 48


def _deg_body(ei_hbm, ew_hbm, out_hbm, col_v, w_v, deg_v):
    cid = lax.axis_index("c")
    sid = lax.axis_index("s")
    wid = sid * _NC + cid
    base = wid * _EPT
    pltpu.sync_copy(ei_hbm.at[1].at[pl.ds(base, _EPT)], col_v)
    pltpu.sync_copy(ew_hbm.at[pl.ds(base, _EPT)], w_v)

    def zero(i, _):
        deg_v[pl.ds(i * 16, 16)] = jnp.zeros((16,), jnp.float32)
        return 0

    lax.fori_loop(0, N_NODES // 16, zero, 0)

    def acc(i, _):
        sl = pl.ds(i * 16, 16)
        plsc.addupdate_scatter(deg_v, [col_v[sl]], w_v[sl])
        return 0

    lax.fori_loop(0, _EPT // 16, acc, 0)
    pltpu.sync_copy(deg_v, out_hbm.at[wid])


def _deg_partials(ei, ew):
    mesh = plsc.VectorSubcoreMesh(core_axis_name="c", subcore_axis_name="s")
    return pl.kernel(
        _deg_body,
        out_type=jax.ShapeDtypeStruct((_NW, N_NODES), jnp.float32),
        mesh=mesh,
        compiler_params=pltpu.CompilerParams(needs_layout_passes=False,
                                             use_tc_tiling_on_sc=False),
        scratch_types=[
            pltpu.VMEM((_EPT,), jnp.int32),
            pltpu.VMEM((_EPT,), jnp.float32),
            pltpu.VMEM((N_NODES,), jnp.float32),
        ],
    )(ei, ew)


def _mm_body(x_ref, w_ref, h_ref):
    h = jnp.dot(x_ref[...], w_ref[...], preferred_element_type=jnp.float32)
    h_ref[0] = h[:, :_HF]
    h_ref[1] = h[:, _HF:]


def _tc_mm(x, w1t):
    nb = 10
    blk = N_NODES // nb
    return pl.pallas_call(
        _mm_body,
        grid=(nb,),
        in_specs=[
            pl.BlockSpec((blk, D_FEAT), lambda i: (i, 0)),
            pl.BlockSpec((D_FEAT, CONV_H), lambda i: (0, 0)),
        ],
        out_specs=pl.BlockSpec((_NC, blk, _HF), lambda i: (0, i, 0)),
        out_shape=jax.ShapeDtypeStruct((_NC, N_NODES, _HF), jnp.float32),
    )(x, w1t)


def _dinv_body(degp_ref, dinv_ref):
    deg = jnp.sum(degp_ref[...], axis=0, keepdims=True) + 1.0
    dinv_ref[...] = lax.rsqrt(deg)


def _tc_dinv(degp):
    return pl.pallas_call(
        _dinv_body,
        out_shape=jax.ShapeDtypeStruct((1, N_NODES), jnp.float32),
    )(degp)


def _msg_body(h_hbm, row_hbm, col_hbm, ew_hbm, dinv_hbm, zero_hbm, batch_hbm,
              b1_hbm, pool_hbm, dinv_v, nrm_v, bv, accg, b1v, *rest):
    n = _NBUF
    ridx = rest[0:n]
    cidx = rest[n:2 * n]
    wch = rest[2 * n:3 * n]
    rows = rest[3 * n:4 * n]
    esem = rest[4 * n:5 * n]
    gsem = rest[5 * n:6 * n]
    ssem = rest[6 * n:7 * n]
    accum = rest[7 * n]

    cid = lax.axis_index("c")
    sid = lax.axis_index("s")
    cbase = sid * _CPT
    hsrc = h_hbm.at[cid]

    pltpu.sync_copy(dinv_hbm, dinv_v)
    pltpu.sync_copy(b1_hbm.at[pl.ds(cid * _HF, _HF)], b1v)

    @pl.when(sid == 0)
    def _():
        pltpu.sync_copy(zero_hbm, accum)

    plsc.subcore_barrier()

    def stage(c, s):
        pltpu.async_copy(row_hbm.at[c], ridx[s], esem[s])
        pltpu.async_copy(col_hbm.at[c], cidx[s], esem[s])
        pltpu.async_copy(ew_hbm.at[c], wch[s], esem[s])

    def ewait(s):
        pltpu.make_async_copy(row_hbm.at[0], ridx[s], esem[s]).wait()
        pltpu.make_async_copy(col_hbm.at[0], cidx[s], esem[s]).wait()
        pltpu.make_async_copy(ew_hbm.at[0], wch[s], esem[s]).wait()

    def gstart(s):
        pltpu.async_copy(hsrc.at[ridx[s]], rows[s], gsem[s])

    def gwait(s):
        pltpu.make_async_copy(hsrc.at[pl.ds(0, _CH)], rows[s],
                              gsem[s]).wait()

    def swait(s):
        pltpu.make_async_copy(rows[s], accum.at[pl.ds(0, _CH)],
                              ssem[s]).wait()

    def compute_scatter(s):
        for k in range(_GRP):
            sl = pl.ds(k * 16, 16)
            dr = plsc.load_gather(dinv_v, [ridx[s][sl]])
            dc = plsc.load_gather(dinv_v, [cidx[s][sl]])
            nrm_v[sl] = dr * wch[s][sl] * dc

        def scale(g, _):
            nrm16 = nrm_v[pl.ds(g * 16, 16)]
            for i16 in range(16):
                ei = g * 16 + i16
                sc = nrm16[i16]
                for f in range(_HF // 16):
                    fs = pl.ds(f * 16, 16)
                    rows[s][ei, fs] = rows[s][ei, fs] * sc
            return 0

        lax.fori_loop(0, _GRP, scale, 0)
        pltpu.async_copy(rows[s], accum.at[cidx[s]], ssem[s], add=True)

    for s in range(_NBUF - 1):
        stage(cbase + s, s)
    for s in range(_NBUF - 2):
        ewait(s)
        gstart(s)

    def outer(t, _):
        for b in range(n):
            j = t * n + b
            gwait(b)
            compute_scatter(b)
            s4 = (b + 4) % n

            @pl.when(j + 4 < _CPT)
            def _():
                @pl.when(j >= 1)
                def _():
                    swait(s4)

                stage(cbase + j + 4, s4)

            s3 = (b + 3) % n

            @pl.when(j + 3 < _CPT)
            def _():
                ewait(s3)
                gstart(s3)

        return 0

    lax.fori_loop(0, _CPT // n, outer, 0)

    for b in range(n):
        swait(b)

    plsc.subcore_barrier()

    def zacc(g, _):
        for f in range(_HF // 16):
            accg[g, pl.ds(f * 16, 16)] = jnp.full((16,), -jnp.inf,
                                                  jnp.float32)
        return 0

    lax.fori_loop(0, NUM_GRAPHS, zacc, 0)

    rb = sid * _PR
    abuf = rows[0]
    hbuf = rows[1]

    def pchunk(c, _):
        base = rb + c * _PCH
        sl = pl.ds(base, _PCH)
        pltpu.sync_copy(accum.at[sl], abuf.at[pl.ds(0, _PCH)])
        pltpu.sync_copy(hsrc.at[sl], hbuf.at[pl.ds(0, _PCH)])
        pltpu.sync_copy(batch_hbm.at[sl], bv)

        def pgrp(t, _):
            o16 = t * 16
            d16 = dinv_v[pl.ds(base + o16, 16)]
            b16 = bv[pl.ds(o16, 16)]
            for i in range(16):
                r = o16 + i
                dsc = d16[i]
                gid = b16[i]
                for f in range(_HF // 16):
                    fs = pl.ds(f * 16, 16)
                    v = abuf[r, fs] + (dsc * dsc) * hbuf[r, fs] + b1v[fs]
                    v = jnp.maximum(v, 0.0)
                    accg[gid, fs] = jnp.maximum(accg[gid, fs], v)
            return 0

        lax.fori_loop(0, _PCH // 16, pgrp, 0)
        return 0

    lax.fori_loop(0, _PR // _PCH, pchunk, 0)

    @pl.when(sid == _NS - 1)
    def _():
        tl = pl.ds(N_NODES - 16, 16)
        pltpu.sync_copy(accum.at[tl], abuf.at[pl.ds(0, 16)])
        pltpu.sync_copy(hsrc.at[tl], hbuf.at[pl.ds(0, 16)])
        pltpu.sync_copy(batch_hbm.at[tl], bv.at[pl.ds(0, 16)])
        d16 = dinv_v[tl]
        b16 = bv[pl.ds(0, 16)]
        for i in range(16):
            dsc = d16[i]
            gid = b16[i]
            for f in range(_HF // 16):
                fs = pl.ds(f * 16, 16)
                v = abuf[i, fs] + (dsc * dsc) * hbuf[i, fs] + b1v[fs]
                v = jnp.maximum(v, 0.0)
                accg[gid, fs] = jnp.maximum(accg[gid, fs], v)

    pltpu.sync_copy(accg, pool_hbm.at[cid].at[sid])


def _msg_pool(h2, row2d, col2d, ew2d, dinv, zeros, batch, b1):
    mesh = plsc.VectorSubcoreMesh(core_axis_name="c", subcore_axis_name="s")
    scratch = [
        pltpu.VMEM((N_NODES,), jnp.float32),
        pltpu.VMEM((_CH,), jnp.float32),
        pltpu.VMEM((_PCH,), jnp.int32),
        pltpu.VMEM((NUM_GRAPHS, _HF), jnp.float32),
        pltpu.VMEM((_HF,), jnp.float32),
    ]
    scratch += [pltpu.VMEM((_CH,), jnp.int32) for _ in range(_NBUF)]
    scratch += [pltpu.VMEM((_CH,), jnp.int32) for _ in range(_NBUF)]
    scratch += [pltpu.VMEM((_CH,), jnp.float32) for _ in range(_NBUF)]
    scratch += [pltpu.VMEM((_CH, _HF), jnp.float32) for _ in range(_NBUF)]
    scratch += [pltpu.SemaphoreType.DMA for _ in range(3 * _NBUF)]
    scratch += [pltpu.VMEM_SHARED((N_NODES, _HF), jnp.float32)]
    return pl.kernel(
        _msg_body,
        out_type=jax.ShapeDtypeStruct((_NC, _NS, NUM_GRAPHS, _HF),
                                      jnp.float32),
        mesh=mesh,
        compiler_params=pltpu.CompilerParams(needs_layout_passes=False,
                                             use_tc_tiling_on_sc=False),
        scratch_types=scratch,
    )(h2, row2d, col2d, ew2d, dinv, zeros, batch, b1)


def _head_body(pm_ref, l1wt_ref, l1b_ref, l2wt_ref, l2b_ref, out_ref):
    p0 = jnp.max(pm_ref[0], axis=0)
    p1 = jnp.max(pm_ref[1], axis=0)
    p = jnp.concatenate([p0, p1], axis=-1)
    z = jnp.dot(p, l1wt_ref[...], preferred_element_type=jnp.float32)
    z = jnp.maximum(z + l1b_ref[...], 0.0)
    out_ref[...] = (jnp.dot(z, l2wt_ref[...],
                            preferred_element_type=jnp.float32)
                    + l2b_ref[...])


def _tc_head(pm, l1wt, l1b, l2wt, l2b):
    return pl.pallas_call(
        _head_body,
        out_shape=jax.ShapeDtypeStruct((NUM_GRAPHS, NUM_CLASSES),
                                       jnp.float32),
    )(pm, l1wt, l1b, l2wt, l2b)


def kernel(x, edge_index, edge_weight, batch, W1, b1, lin1_w, lin1_b,
           lin2_w, lin2_b):
    ei = edge_index.astype(jnp.int32)
    ew = jnp.ravel(edge_weight).astype(jnp.float32)
    x = x.astype(jnp.float32)

    degp = _deg_partials(ei, ew)
    h2 = _tc_mm(x, W1.T)
    dinv = _tc_dinv(degp).reshape(N_NODES)
    zeros = jnp.zeros((N_NODES, _HF), jnp.float32)
    pm = _msg_pool(h2, ei[0].reshape(_NCHUNK, _CH),
                   ei[1].reshape(_NCHUNK, _CH),
                   ew.reshape(_NCHUNK, _CH), dinv, zeros,
                   batch.astype(jnp.int32), b1)
    return _tc_head(pm, lin1_w.T, lin1_b.reshape(1, LIN_H),
                    lin2_w.T, lin2_b.reshape(1, NUM_CLASSES))


# speedup vs baseline: 1.4947x; 1.4940x over previous
import jax
import jax.numpy as jnp
from jax import lax
from jax.experimental import pallas as pl
from jax.experimental.pallas import tpu as pltpu
from jax.experimental.pallas import tpu_sc as plsc

N_NODES = 10000
N_EDGES = 320000
D_FEAT = 128
CONV_H = 128
LIN_H = 64
NUM_CLASSES = 10
NUM_GRAPHS = 64

_NC = 2
_NS = 16
_NW = _NC * _NS
_EPT = N_EDGES // _NW
_CH = 80
_NCHUNK = N_EDGES // _CH
_CPT = _NCHUNK // _NS
_NBUF = 5
_HF = CONV_H // 2
_GRP = _CH // 16
_RPT = N_NODES // _NS


def _deg_body(ei_hbm, ew_hbm, out_hbm, col_v, w_v, deg_v):
    cid = lax.axis_index("c")
    sid = lax.axis_index("s")
    wid = sid * _NC + cid
    base = wid * _EPT
    pltpu.sync_copy(ei_hbm.at[1].at[pl.ds(base, _EPT)], col_v)
    pltpu.sync_copy(ew_hbm.at[pl.ds(base, _EPT)], w_v)

    def zero(i, _):
        deg_v[pl.ds(i * 16, 16)] = jnp.zeros((16,), jnp.float32)
        return 0

    lax.fori_loop(0, N_NODES // 16, zero, 0)

    def acc(i, _):
        sl = pl.ds(i * 16, 16)
        plsc.addupdate_scatter(deg_v, [col_v[sl]], w_v[sl])
        return 0

    lax.fori_loop(0, _EPT // 16, acc, 0)
    pltpu.sync_copy(deg_v, out_hbm.at[wid])


def _deg_partials(ei, ew):
    mesh = plsc.VectorSubcoreMesh(core_axis_name="c", subcore_axis_name="s")
    return pl.kernel(
        _deg_body,
        out_type=jax.ShapeDtypeStruct((_NW, N_NODES), jnp.float32),
        mesh=mesh,
        compiler_params=pltpu.CompilerParams(needs_layout_passes=False,
                                             use_tc_tiling_on_sc=False),
        scratch_types=[
            pltpu.VMEM((_EPT,), jnp.int32),
            pltpu.VMEM((_EPT,), jnp.float32),
            pltpu.VMEM((N_NODES,), jnp.float32),
        ],
    )(ei, ew)


def _mm_body(x_ref, w_ref, h_ref):
    h = jnp.dot(x_ref[...], w_ref[...], preferred_element_type=jnp.float32)
    h_ref[0] = h[:, :_HF]
    h_ref[1] = h[:, _HF:]


def _tc_mm(x, w1t):
    nb = 10
    blk = N_NODES // nb
    return pl.pallas_call(
        _mm_body,
        grid=(nb,),
        in_specs=[
            pl.BlockSpec((blk, D_FEAT), lambda i: (i, 0)),
            pl.BlockSpec((D_FEAT, CONV_H), lambda i: (0, 0)),
        ],
        out_specs=pl.BlockSpec((_NC, blk, _HF), lambda i: (0, i, 0)),
        out_shape=jax.ShapeDtypeStruct((_NC, N_NODES, _HF), jnp.float32),
    )(x, w1t)


def _dinv_body(degp_ref, dinv_ref):
    deg = jnp.sum(degp_ref[...], axis=0, keepdims=True) + 1.0
    dinv_ref[...] = lax.rsqrt(deg)


def _tc_dinv(degp):
    return pl.pallas_call(
        _dinv_body,
        out_shape=jax.ShapeDtypeStruct((1, N_NODES), jnp.float32),
    )(degp)


def _msg_body(h_hbm, row_hbm, col_hbm, ew_hbm, dinv_hbm, zero_hbm, part_hbm,
              dinv_v, nrm_v, *rest):
    n = _NBUF
    ridx = rest[0:n]
    cidx = rest[n:2 * n]
    wch = rest[2 * n:3 * n]
    rows = rest[3 * n:4 * n]
    esem = rest[4 * n:5 * n]
    gsem = rest[5 * n:6 * n]
    ssem = rest[6 * n:7 * n]
    accum = rest[7 * n]

    cid = lax.axis_index("c")
    sid = lax.axis_index("s")
    cbase = sid * _CPT
    hsrc = h_hbm.at[cid]

    pltpu.sync_copy(dinv_hbm, dinv_v)

    @pl.when(sid == 0)
    def _():
        pltpu.sync_copy(zero_hbm, accum)

    plsc.subcore_barrier()

    def stage(c, s):
        pltpu.async_copy(row_hbm.at[c], ridx[s], esem[s])
        pltpu.async_copy(col_hbm.at[c], cidx[s], esem[s])
        pltpu.async_copy(ew_hbm.at[c], wch[s], esem[s])

    def ewait(s):
        pltpu.make_async_copy(row_hbm.at[0], ridx[s], esem[s]).wait()
        pltpu.make_async_copy(col_hbm.at[0], cidx[s], esem[s]).wait()
        pltpu.make_async_copy(ew_hbm.at[0], wch[s], esem[s]).wait()

    def gstart(s):
        pltpu.async_copy(hsrc.at[ridx[s]], rows[s], gsem[s])

    def gwait(s):
        pltpu.make_async_copy(hsrc.at[pl.ds(0, _CH)], rows[s],
                              gsem[s]).wait()

    def swait(s):
        pltpu.make_async_copy(rows[s], accum.at[pl.ds(0, _CH)],
                              ssem[s]).wait()

    def compute_scatter(s):
        for k in range(_GRP):
            sl = pl.ds(k * 16, 16)
            dr = plsc.load_gather(dinv_v, [ridx[s][sl]])
            dc = plsc.load_gather(dinv_v, [cidx[s][sl]])
            nrm_v[sl] = dr * wch[s][sl] * dc

        def scale(g, _):
            nrm16 = nrm_v[pl.ds(g * 16, 16)]
            for i16 in range(16):
                ei = g * 16 + i16
                sc = nrm16[i16]
                for f in range(_HF // 16):
                    fs = pl.ds(f * 16, 16)
                    rows[s][ei, fs] = rows[s][ei, fs] * sc
            return 0

        lax.fori_loop(0, _GRP, scale, 0)
        pltpu.async_copy(rows[s], accum.at[cidx[s]], ssem[s], add=True)

    for s in range(_NBUF - 1):
        stage(cbase + s, s)
    for s in range(_NBUF - 2):
        ewait(s)
        gstart(s)

    def outer(t, _):
        for b in range(n):
            j = t * n + b
            gwait(b)
            compute_scatter(b)
            s4 = (b + 4) % n

            @pl.when(j + 4 < _CPT)
            def _():
                @pl.when(j >= 1)
                def _():
                    swait(s4)

                stage(cbase + j + 4, s4)

            s3 = (b + 3) % n

            @pl.when(j + 3 < _CPT)
            def _():
                ewait(s3)
                gstart(s3)

        return 0

    lax.fori_loop(0, _CPT // n, outer, 0)

    for b in range(n):
        swait(b)

    plsc.subcore_barrier()

    rb = sid * _RPT
    pltpu.sync_copy(accum.at[pl.ds(rb, _RPT)],
                    part_hbm.at[cid].at[pl.ds(rb, _RPT)])


def _msg_partials(h2, row2d, col2d, ew2d, dinv, zeros):
    mesh = plsc.VectorSubcoreMesh(core_axis_name="c", subcore_axis_name="s")
    scratch = [
        pltpu.VMEM((N_NODES,), jnp.float32),
        pltpu.VMEM((_CH,), jnp.float32),
    ]
    scratch += [pltpu.VMEM((_CH,), jnp.int32) for _ in range(_NBUF)]
    scratch += [pltpu.VMEM((_CH,), jnp.int32) for _ in range(_NBUF)]
    scratch += [pltpu.VMEM((_CH,), jnp.float32) for _ in range(_NBUF)]
    scratch += [pltpu.VMEM((_CH, _HF), jnp.float32) for _ in range(_NBUF)]
    scratch += [pltpu.SemaphoreType.DMA for _ in range(3 * _NBUF)]
    scratch += [pltpu.VMEM_SHARED((N_NODES, _HF), jnp.float32)]
    return pl.kernel(
        _msg_body,
        out_type=jax.ShapeDtypeStruct((_NC, N_NODES, _HF), jnp.float32),
        mesh=mesh,
        compiler_params=pltpu.CompilerParams(needs_layout_passes=False,
                                             use_tc_tiling_on_sc=False),
        scratch_types=scratch,
    )(h2, row2d, col2d, ew2d, dinv, zeros)


def _tc2_body(part_ref, h_ref, dinv_ref, b1_ref, batch_ref,
              l1wt_ref, l1b_ref, l2wt_ref, l2b_ref, out_ref, pool_acc):
    i = pl.program_id(0)
    d = dinv_ref[...]
    p = jnp.concatenate([part_ref[0], part_ref[1]], axis=-1)
    h = jnp.concatenate([h_ref[0], h_ref[1]], axis=-1)
    o = p + h * (d * d) + b1_ref[...]
    o = jnp.maximum(o, 0.0)
    bid = batch_ref[...]

    @pl.when(i == 0)
    def _():
        pool_acc[...] = jnp.full((NUM_GRAPHS, CONV_H), -jnp.inf,
                                 dtype=jnp.float32)

    for g in range(NUM_GRAPHS):
        og = jnp.where(bid == g, o, -jnp.inf)
        cg = jnp.max(og, axis=0, keepdims=True)
        sl = pl.ds(g, 1)
        pool_acc[sl, :] = jnp.maximum(pool_acc[sl, :], cg)

    @pl.when(i == pl.num_programs(0) - 1)
    def _():
        pm = pool_acc[...]
        z = jnp.dot(pm, l1wt_ref[...], preferred_element_type=jnp.float32)
        z = jnp.maximum(z + l1b_ref[...], 0.0)
        out_ref[...] = (jnp.dot(z, l2wt_ref[...],
                                preferred_element_type=jnp.float32)
                        + l2b_ref[...])


def _tc2(part, h, dinv_col, b1, batch_col, l1wt, l1b, l2wt, l2b):
    nb = 10
    blk = N_NODES // nb
    return pl.pallas_call(
        _tc2_body,
        grid=(nb,),
        in_specs=[
            pl.BlockSpec((_NC, blk, _HF), lambda i: (0, i, 0)),
            pl.BlockSpec((_NC, blk, _HF), lambda i: (0, i, 0)),
            pl.BlockSpec((blk, 1), lambda i: (i, 0)),
            pl.BlockSpec((1, CONV_H), lambda i: (0, 0)),
            pl.BlockSpec((blk, 1), lambda i: (i, 0)),
            pl.BlockSpec((CONV_H, LIN_H), lambda i: (0, 0)),
            pl.BlockSpec((1, LIN_H), lambda i: (0, 0)),
            pl.BlockSpec((LIN_H, NUM_CLASSES), lambda i: (0, 0)),
            pl.BlockSpec((1, NUM_CLASSES), lambda i: (0, 0)),
        ],
        out_specs=pl.BlockSpec((NUM_GRAPHS, NUM_CLASSES), lambda i: (0, 0)),
        out_shape=jax.ShapeDtypeStruct((NUM_GRAPHS, NUM_CLASSES), jnp.float32),
        scratch_shapes=[pltpu.VMEM((NUM_GRAPHS, CONV_H), jnp.float32)],
    )(part, h, dinv_col, b1, batch_col, l1wt, l1b, l2wt, l2b)


def kernel(x, edge_index, edge_weight, batch, W1, b1, lin1_w, lin1_b,
           lin2_w, lin2_b):
    ei = edge_index.astype(jnp.int32)
    ew = jnp.ravel(edge_weight).astype(jnp.float32)
    x = x.astype(jnp.float32)

    degp = _deg_partials(ei, ew)
    h2 = _tc_mm(x, W1.T)
    dinv = _tc_dinv(degp).reshape(N_NODES)
    zeros = jnp.zeros((N_NODES, _HF), jnp.float32)
    part = _msg_partials(h2, ei[0].reshape(_NCHUNK, _CH),
                         ei[1].reshape(_NCHUNK, _CH),
                         ew.reshape(_NCHUNK, _CH), dinv, zeros)
    return _tc2(part, h2, dinv.reshape(N_NODES, 1), b1.reshape(1, CONV_H),
                batch.reshape(N_NODES, 1).astype(jnp.int32),
                lin1_w.T, lin1_b.reshape(1, LIN_H),
                lin2_w.T, lin2_b.reshape(1, NUM_CLASSES))
